# baseline (speedup 1.0000x reference)
import functools

import jax
import jax.numpy as jnp
from jax import lax
from jax.experimental import pallas as pl
from jax.experimental.pallas import tpu as pltpu
from jax.experimental.pallas import tpu_sc as plsc

R = 128
N = 32768
F = 4
J = 5
NC = 2
NS = 16
L = 16
NW = NC * NS
RPW = R // NW
C = 4096
C4 = C * F
NCH = N // C
NEG = -3.0e38


def _bf16_rne(v):
    u = plsc.bitcast(v, jnp.int32)
    r = (u + 0x7FFF + ((u >> 16) & 1)) & jnp.int32(-65536)
    return plsc.bitcast(r, jnp.float32)


def _sc_topk(xf, const):
    mesh = plsc.VectorSubcoreMesh(
        core_axis_name="c", subcore_axis_name="s",
        num_cores=NC, num_subcores=NS)

    @functools.partial(
        pl.kernel,
        out_type=(jax.ShapeDtypeStruct((R, L), jnp.float32),
                  jax.ShapeDtypeStruct((R, L), jnp.int32)),
        mesh=mesh,
        compiler_params=pltpu.CompilerParams(needs_layout_passes=False),
        scratch_types=[
            pltpu.VMEM((C4,), jnp.float32),
            pltpu.VMEM((C4,), jnp.float32),
            pltpu.VMEM((32, L), jnp.float32),
            pltpu.VMEM((4, L), jnp.float32),
            pltpu.VMEM((4, L), jnp.int32),
            pltpu.VMEM((RPW, L), jnp.float32),
            pltpu.VMEM((RPW, L), jnp.int32),
            pltpu.SemaphoreType.DMA,
            pltpu.SemaphoreType.DMA,
        ],
    )
    def k(x_hbm, c_hbm, ov_hbm, oi_hbm,
          buf0, buf1, cv, mv, mi, ov, oi, sem0, sem1):
        wid = lax.axis_index("s") * NC + lax.axis_index("c")
        pltpu.sync_copy(c_hbm, cv)

        wrow = [[cv[j * F + f, :] for f in range(F)] for j in range(J)]
        wvec = [cv[J * F + f, :] for f in range(F)]
        brow = [cv[J * F + F + j, :] for j in range(J)]
        iota = lax.iota(jnp.int32, L)
        pos0 = iota * F
        neg = jnp.full((L,), NEG, jnp.float32)
        zi = jnp.zeros((L,), jnp.int32)
        grow = jnp.minimum(iota // 3, 3)
        gcol = iota - grow * 3

        bufs = (buf0, buf1)
        sems = (sem0, sem1)

        def make_body(buf):
            def body(_, carry):
                t1, t2, t3, i1, i2, i3, cand, pos = carry
                f0 = plsc.load_gather(buf, [pos])
                f1 = plsc.load_gather(buf, [pos + 1])
                f2 = plsc.load_gather(buf, [pos + 2])
                f3 = plsc.load_gather(buf, [pos + 3])
                t = [_bf16_rne(f + wk) for f, wk in
                     zip((f0, f1, f2, f3), wvec)]
                s = jnp.zeros((L,), jnp.float32)
                for j in range(J):
                    a = (t[0] * wrow[j][0] + t[1] * wrow[j][1]
                         + t[2] * wrow[j][2] + t[3] * wrow[j][3]
                         + brow[j])
                    s = s + jnp.maximum(a, 0.0)
                c1 = s > t1
                c2 = s > t2
                c3 = s > t3
                t3n = jnp.where(c2, t2, jnp.where(c3, s, t3))
                i3n = jnp.where(c2, i2, jnp.where(c3, cand, i3))
                t2n = jnp.where(c1, t1, jnp.where(c2, s, t2))
                i2n = jnp.where(c1, i1, jnp.where(c2, cand, i2))
                t1n = jnp.where(c1, s, t1)
                i1n = jnp.where(c1, cand, i1)
                return (t1n, t2n, t3n, i1n, i2n, i3n, cand + L, pos + L * F)
            return body

        for r in range(RPW):
            row = wid * RPW + r
            cp = pltpu.async_copy(x_hbm.at[row, pl.ds(0, C4)], buf0, sem0)
            t1 = neg; t2 = neg; t3 = neg
            i1 = zi; i2 = zi; i3 = zi
            cand = iota
            for ch in range(NCH):
                nxt = None
                if ch + 1 < NCH:
                    nxt = pltpu.async_copy(
                        x_hbm.at[row, pl.ds((ch + 1) * C4, C4)],
                        bufs[(ch + 1) % 2], sems[(ch + 1) % 2])
                cp.wait()
                carry = (t1, t2, t3, i1, i2, i3, cand, pos0)
                t1, t2, t3, i1, i2, i3, cand, _ = lax.fori_loop(
                    0, C // L, make_body(bufs[ch % 2]), carry)
                cp = nxt

            s1k, s1v = plsc.sort_key_val(t1, i1, descending=True)
            s2k, s2v = plsc.sort_key_val(t2, i2, descending=True)
            s3k, s3v = plsc.sort_key_val(t3, i3, descending=True)
            mv[0, :] = s1k
            mv[1, :] = s2k
            mv[2, :] = s3k
            mv[3, :] = neg
            mi[0, :] = s1v
            mi[1, :] = s2v
            mi[2, :] = s3v
            mi[3, :] = zi
            gv = plsc.load_gather(mv, [grow, gcol])
            gi = plsc.load_gather(mi, [grow, gcol])
            fk, fi = plsc.sort_key_val(gv, gi, descending=True)
            ov[r, :] = fk
            oi[r, :] = fi

        pltpu.sync_copy(ov, ov_hbm.at[pl.ds(wid * RPW, RPW)])
        pltpu.sync_copy(oi, oi_hbm.at[pl.ds(wid * RPW, RPW)])

    return k(xf, const)


def kernel(x, W, b):
    w = jnp.mean(W, axis=0)
    u = jax.lax.bitcast_convert_type(W, jnp.int32)
    r = (u + 0x7FFF + ((u >> 16) & 1)) & jnp.int32(-65536)
    wb = jax.lax.bitcast_convert_type(r, jnp.float32)
    flat = jnp.concatenate([wb.reshape(-1), w, b,
                            jnp.zeros((32 - J * F - F - J,), jnp.float32)])
    const = jnp.broadcast_to(flat[:, None], (32, L))
    xf = x.reshape(R, N * F)
    ov, oi = _sc_topk(xf, const)
    return ov[:, :3], oi[:, :3]
